# baseline (speedup 1.0000x reference)
import functools

import jax
import jax.numpy as jnp
from jax import lax
from jax.experimental import pallas as pl
from jax.experimental.pallas import tpu as pltpu

N = 10000
E = 160000
D_IN = 256
H = 512
DROP_EDGE = 0.2
DROP_FEAT = 0.2

_LANE = 128
_BR = 1000
_NRB = N // _BR

_NSC = 2
_NSUB = 16
_NACC = 10240
_EBATCH = 128
_EPAD = 131072


def _tables_body(x_ref, dinv_ref, o_ref):
    o_ref[0] = x_ref[...] * dinv_ref[...]


def _make_table(x, dinv):
    nc = D_IN // _LANE
    return pl.pallas_call(
        _tables_body,
        grid=(nc, _NRB),
        in_specs=[
            pl.BlockSpec((_BR, _LANE), lambda c, r: (r, c)),
            pl.BlockSpec((_BR, 1), lambda c, r: (r, 0)),
        ],
        out_specs=pl.BlockSpec((1, _BR, _LANE), lambda c, r: (c, r, 0)),
        out_shape=jax.ShapeDtypeStruct((nc, N, _LANE), jnp.float32),
    )(x, dinv)


def _mid_body(s_ref, xs_ref, dinv_ref, w1_ref, w2_ref, b1_ref, a1_ref, o_ref):
    nc_in = s_ref.shape[0]
    ax = jnp.concatenate(
        [s_ref[c] + xs_ref[c] for c in range(nc_in)], axis=1
    ) * dinv_ref[...]
    hpre = jnp.dot(ax, w1_ref[...], preferred_element_type=jnp.float32) + b1_ref[0]
    a1 = a1_ref[0]
    h = jnp.where(hpre >= 0.0, hpre, a1 * hpre)
    hs = jnp.dot(h, w2_ref[...], preferred_element_type=jnp.float32) * dinv_ref[...]
    nc_out = o_ref.shape[0]
    for c in range(nc_out):
        o_ref[c] = hs[:, c * _LANE:(c + 1) * _LANE]


def _mid_layer(S1, xs, dinv, W1v, W2, b1, a1):
    nc_in = D_IN // _LANE
    nc_out = H // _LANE
    return pl.pallas_call(
        _mid_body,
        grid=(_NRB,),
        in_specs=[
            pl.BlockSpec((nc_in, _BR, _LANE), lambda r: (0, r, 0)),
            pl.BlockSpec((nc_in, _BR, _LANE), lambda r: (0, r, 0)),
            pl.BlockSpec((_BR, 1), lambda r: (r, 0)),
            pl.BlockSpec((D_IN, H), lambda r: (0, 0)),
            pl.BlockSpec((H, H), lambda r: (0, 0)),
            pl.BlockSpec((1, H), lambda r: (0, 0)),
            pl.BlockSpec((1, 1), lambda r: (0, 0)),
        ],
        out_specs=pl.BlockSpec((nc_out, _BR, _LANE), lambda r: (0, r, 0)),
        out_shape=jax.ShapeDtypeStruct((nc_out, N, _LANE), jnp.float32),
    )(S1, xs, dinv, W1v, W2, b1, a1)


def _post_body(s_ref, hs_ref, dinv_ref, b2_ref, a2_ref, h2_ref, st_ref):
    nc = s_ref.shape[0]
    acc = jnp.concatenate(
        [s_ref[c] + hs_ref[c] for c in range(nc)], axis=1
    ) * dinv_ref[...]
    hpre = acc + b2_ref[0]
    a2 = a2_ref[0]
    h2 = jnp.where(hpre >= 0.0, hpre, a2 * hpre)
    h2_ref[...] = h2
    s = jnp.sum(h2, axis=0, keepdims=True)
    sq = jnp.sum(h2 * h2, axis=0, keepdims=True)
    st = jnp.concatenate([s, sq], axis=0)

    @pl.when(pl.program_id(0) == 0)
    def _init():
        st_ref[...] = st

    @pl.when(pl.program_id(0) != 0)
    def _acc():
        st_ref[...] += st


def _post_layer(S2, hs, dinv, b2, a2):
    nc = H // _LANE
    return pl.pallas_call(
        _post_body,
        grid=(_NRB,),
        in_specs=[
            pl.BlockSpec((nc, _BR, _LANE), lambda r: (0, r, 0)),
            pl.BlockSpec((nc, _BR, _LANE), lambda r: (0, r, 0)),
            pl.BlockSpec((_BR, 1), lambda r: (r, 0)),
            pl.BlockSpec((1, H), lambda r: (0, 0)),
            pl.BlockSpec((1, 1), lambda r: (0, 0)),
        ],
        out_specs=[
            pl.BlockSpec((_BR, H), lambda r: (r, 0)),
            pl.BlockSpec((2, H), lambda r: (0, 0)),
        ],
        out_shape=[
            jax.ShapeDtypeStruct((N, H), jnp.float32),
            jax.ShapeDtypeStruct((2, H), jnp.float32),
        ],
    )(S2, hs, dinv, b2, a2)


def _std_body(h2_ref, mu_ref, sd_ref, o_ref):
    o_ref[...] = (h2_ref[...] - mu_ref[...]) / sd_ref[...]


def _standardize(h2, mu, sd):
    return pl.pallas_call(
        _std_body,
        grid=(_NRB,),
        in_specs=[
            pl.BlockSpec((_BR, H), lambda r: (r, 0)),
            pl.BlockSpec((1, H), lambda r: (0, 0)),
            pl.BlockSpec((1, H), lambda r: (0, 0)),
        ],
        out_specs=pl.BlockSpec((_BR, H), lambda r: (r, 0)),
        out_shape=jax.ShapeDtypeStruct((N, H), jnp.float32),
    )(h2, mu, sd)


def _compact_edges(edge_index, keeps):
    pos = jnp.stack([
        jnp.nonzero(k, size=_EPAD, fill_value=E)[0] for k in keeps
    ])
    valid = pos < E
    pos_c = jnp.minimum(pos, E - 1)
    spread = jnp.broadcast_to(jnp.arange(_EPAD, dtype=jnp.int32), (2, _EPAD))
    src = jnp.where(valid, edge_index[0, pos_c], spread % N).astype(jnp.int32)
    dst = jnp.where(valid, edge_index[1, pos_c],
                    N + (spread % (_NACC - N))).astype(jnp.int32)
    vofs = jnp.array([[0], [N]], jnp.int32)
    idx = jnp.minimum(dst, N - 1) + vofs
    deg = jnp.ones((2 * N,), jnp.float32).at[idx.reshape(-1)].add(
        valid.astype(jnp.float32).reshape(-1))
    dinv = lax.rsqrt(deg).reshape(2, N)
    return src, dst, dinv


def _make_segsum(P, nb):
    from jax.experimental.pallas import tpu_sc as plsc

    mesh = plsc.VectorSubcoreMesh(
        core_axis_name="c", subcore_axis_name="s", num_cores=_NSC,
        num_subcores=_NSUB)
    acc_rows_per_sub = _NACC // _NSUB
    n_drain = acc_rows_per_sub // _EBATCH
    nh = 2
    nb_h = nb // nh

    @functools.partial(
        pl.kernel,
        out_type=jax.ShapeDtypeStruct((P, _NACC, _LANE), jnp.float32),
        mesh=mesh,
        scratch_types=[
            pltpu.VMEM_SHARED((_NACC, _LANE), jnp.float32),
            pltpu.VMEM((nb_h + 8, _EBATCH), jnp.int32),
            pltpu.VMEM((nb_h, _EBATCH), jnp.int32),
            pltpu.VMEM((_EBATCH, _LANE), jnp.float32),
            pltpu.VMEM((_EBATCH, _LANE), jnp.float32),
            pltpu.SemaphoreType.DMA,
            pltpu.SemaphoreType.DMA,
        ],
    )
    def segsum(table_hbm, src_hbm, dst_hbm, zeros_hbm, out_hbm,
               acc, src_v, dst_v, rows0, rows1, sem0, sem1):
        cid = lax.axis_index("c")
        sid = lax.axis_index("s")

        def process_pair(p):
            pltpu.sync_copy(zeros_hbm, rows0)
            for j in range(acc_rows_per_sub // _EBATCH):
                pltpu.sync_copy(
                    rows0,
                    acc.at[pl.ds(sid * acc_rows_per_sub + j * _EBATCH,
                                 _EBATCH)])
            plsc.subcore_barrier()

            tbl = table_hbm.at[p]

            def gather(b, buf, sem):
                pltpu.make_async_copy(tbl.at[src_v.at[b]], buf, sem).start()

            def wait(buf, sem):
                pltpu.make_async_copy(tbl.at[pl.ds(0, _EBATCH)], buf,
                                      sem).wait()

            for hh in range(nh):
                pltpu.sync_copy(
                    src_hbm.at[sid].at[pl.ds(hh * nb_h, nb_h + 8)], src_v)
                pltpu.sync_copy(
                    dst_hbm.at[sid].at[pl.ds(hh * nb_h, nb_h)], dst_v)

                gather(0, rows0, sem0)
                gather(1, rows1, sem1)

                def body(i, carry):
                    b0 = 2 * i
                    wait(rows0, sem0)
                    pltpu.sync_copy(rows0, acc.at[dst_v.at[b0]], add=True)
                    gather(b0 + 2, rows0, sem0)
                    wait(rows1, sem1)
                    pltpu.sync_copy(rows1, acc.at[dst_v.at[b0 + 1]],
                                    add=True)
                    gather(b0 + 3, rows1, sem1)
                    return carry

                lax.fori_loop(0, nb_h // 2, body, 0, unroll=False)
                wait(rows0, sem0)
                wait(rows1, sem1)
            plsc.subcore_barrier()
            for j in range(n_drain):
                r0 = sid * acc_rows_per_sub + j * _EBATCH
                pltpu.sync_copy(acc.at[pl.ds(r0, _EBATCH)], rows0)
                pltpu.sync_copy(rows0, out_hbm.at[p].at[pl.ds(r0, _EBATCH)])
            plsc.subcore_barrier()

        for k in range(_NSC):
            @pl.when(cid == k)
            def _run():
                for p in range(k, P, _NSC):
                    process_pair(p)

    return segsum


def _segsum_sc(table, src, dst):
    P = table.shape[0]
    nb = _EPAD // (_NSUB * _EBATCH)
    src_hbm = src.reshape(_NSUB, nb, _EBATCH)
    pad_src = jnp.arange(_NSUB * 8 * _EBATCH, dtype=jnp.int32) % N
    src_hbm = jnp.concatenate(
        [src_hbm, pad_src.reshape(_NSUB, 8, _EBATCH)], axis=1)
    dst_hbm = dst.reshape(_NSUB, nb, _EBATCH)
    zeros_hbm = jnp.zeros((_EBATCH, _LANE), jnp.float32)
    fn = _make_segsum(P, nb)
    return fn(table, src_hbm, dst_hbm, zeros_hbm)


def kernel(x, edge_index, W1, b1, a1, W2, b2, a2):
    rkey = jax.random.key(42)
    ke1, ke2, kf1, kf2 = jax.random.split(rkey, 4)
    keeps = [jax.random.bernoulli(k, 1.0 - DROP_EDGE, (E,)) for k in (ke1, ke2)]
    fmasks = [jax.random.bernoulli(k, 1.0 - DROP_FEAT, (1, D_IN))
              for k in (kf1, kf2)]

    b1r = b1[None, :]
    b2r = b2[None, :]
    a1r = a1[:, None]
    a2r = a2[:, None]

    srcs, dsts, dinv2 = _compact_edges(edge_index, keeps)
    dinvs = [dinv2[v][:, None] for v in range(2)]
    W1vs = [W1 * fmasks[v][0].astype(jnp.float32)[:, None] for v in range(2)]

    zs = []
    for v in range(2):
        xs = _make_table(x, dinvs[v])
        S1 = _segsum_sc(xs, srcs[v], dsts[v])
        hs = _mid_layer(S1, xs, dinvs[v], W1vs[v], W2, b1r, a1r)
        S2 = _segsum_sc(hs, srcs[v], dsts[v])

        h2, stats = _post_layer(S2, hs, dinvs[v], b2r, a2r)
        mu = stats[0] / N
        var = (stats[1] - N * mu * mu) / (N - 1)
        sd = jnp.sqrt(var)
        zs.append(_standardize(h2, mu[None, :], sd[None, :]))
    return (zs[0], zs[1])


# speedup vs baseline: 1.0575x; 1.0575x over previous
import functools

import jax
import jax.numpy as jnp
from jax import lax
from jax.experimental import pallas as pl
from jax.experimental.pallas import tpu as pltpu

N = 10000
E = 160000
D_IN = 256
H = 512
DROP_EDGE = 0.2
DROP_FEAT = 0.2

_LANE = 128
_BR = 1000
_NRB = N // _BR

_NSC = 2
_NSUB = 16
_NACC = 10240
_EBATCH = 128
_EPAD = 131072


def _tables_body(x_ref, dinv_ref, o_ref):
    o_ref[0] = x_ref[...] * dinv_ref[...]


def _make_table(x, dinv):
    nc = D_IN // _LANE
    return pl.pallas_call(
        _tables_body,
        grid=(nc, _NRB),
        in_specs=[
            pl.BlockSpec((_BR, _LANE), lambda c, r: (r, c)),
            pl.BlockSpec((_BR, 1), lambda c, r: (r, 0)),
        ],
        out_specs=pl.BlockSpec((1, _BR, _LANE), lambda c, r: (c, r, 0)),
        out_shape=jax.ShapeDtypeStruct((nc, N, _LANE), jnp.float32),
    )(x, dinv)


def _mid_body(s_ref, xs_ref, dinv_ref, w1_ref, w2_ref, b1_ref, a1_ref, o_ref):
    nc_in = s_ref.shape[0]
    ax = jnp.concatenate(
        [s_ref[c] + xs_ref[c] for c in range(nc_in)], axis=1
    ) * dinv_ref[...]
    hpre = jnp.dot(ax, w1_ref[...], preferred_element_type=jnp.float32) + b1_ref[0]
    a1 = a1_ref[0]
    h = jnp.where(hpre >= 0.0, hpre, a1 * hpre)
    hs = jnp.dot(h, w2_ref[...], preferred_element_type=jnp.float32) * dinv_ref[...]
    nc_out = o_ref.shape[0]
    for c in range(nc_out):
        o_ref[c] = hs[:, c * _LANE:(c + 1) * _LANE]


def _mid_layer(S1, xs, dinv, W1v, W2, b1, a1):
    nc_in = D_IN // _LANE
    nc_out = H // _LANE
    return pl.pallas_call(
        _mid_body,
        grid=(_NRB,),
        in_specs=[
            pl.BlockSpec((nc_in, _BR, _LANE), lambda r: (0, r, 0)),
            pl.BlockSpec((nc_in, _BR, _LANE), lambda r: (0, r, 0)),
            pl.BlockSpec((_BR, 1), lambda r: (r, 0)),
            pl.BlockSpec((D_IN, H), lambda r: (0, 0)),
            pl.BlockSpec((H, H), lambda r: (0, 0)),
            pl.BlockSpec((1, H), lambda r: (0, 0)),
            pl.BlockSpec((1, 1), lambda r: (0, 0)),
        ],
        out_specs=pl.BlockSpec((nc_out, _BR, _LANE), lambda r: (0, r, 0)),
        out_shape=jax.ShapeDtypeStruct((nc_out, N, _LANE), jnp.float32),
    )(S1, xs, dinv, W1v, W2, b1, a1)


def _post_body(s_ref, hs_ref, dinv_ref, b2_ref, a2_ref, h2_ref, st_ref):
    nc = s_ref.shape[0]
    acc = jnp.concatenate(
        [s_ref[c] + hs_ref[c] for c in range(nc)], axis=1
    ) * dinv_ref[...]
    hpre = acc + b2_ref[0]
    a2 = a2_ref[0]
    h2 = jnp.where(hpre >= 0.0, hpre, a2 * hpre)
    h2_ref[...] = h2
    s = jnp.sum(h2, axis=0, keepdims=True)
    sq = jnp.sum(h2 * h2, axis=0, keepdims=True)
    st = jnp.concatenate([s, sq], axis=0)

    @pl.when(pl.program_id(0) == 0)
    def _init():
        st_ref[...] = st

    @pl.when(pl.program_id(0) != 0)
    def _acc():
        st_ref[...] += st


def _post_layer(S2, hs, dinv, b2, a2):
    nc = H // _LANE
    return pl.pallas_call(
        _post_body,
        grid=(_NRB,),
        in_specs=[
            pl.BlockSpec((nc, _BR, _LANE), lambda r: (0, r, 0)),
            pl.BlockSpec((nc, _BR, _LANE), lambda r: (0, r, 0)),
            pl.BlockSpec((_BR, 1), lambda r: (r, 0)),
            pl.BlockSpec((1, H), lambda r: (0, 0)),
            pl.BlockSpec((1, 1), lambda r: (0, 0)),
        ],
        out_specs=[
            pl.BlockSpec((_BR, H), lambda r: (r, 0)),
            pl.BlockSpec((2, H), lambda r: (0, 0)),
        ],
        out_shape=[
            jax.ShapeDtypeStruct((N, H), jnp.float32),
            jax.ShapeDtypeStruct((2, H), jnp.float32),
        ],
    )(S2, hs, dinv, b2, a2)


def _std_body(h2_ref, mu_ref, sd_ref, o_ref):
    o_ref[...] = (h2_ref[...] - mu_ref[...]) / sd_ref[...]


def _standardize(h2, mu, sd):
    return pl.pallas_call(
        _std_body,
        grid=(_NRB,),
        in_specs=[
            pl.BlockSpec((_BR, H), lambda r: (r, 0)),
            pl.BlockSpec((1, H), lambda r: (0, 0)),
            pl.BlockSpec((1, H), lambda r: (0, 0)),
        ],
        out_specs=pl.BlockSpec((_BR, H), lambda r: (r, 0)),
        out_shape=jax.ShapeDtypeStruct((N, H), jnp.float32),
    )(h2, mu, sd)


def _compact_edges(edge_index, keep):
    pos = jnp.nonzero(keep, size=_EPAD, fill_value=E)[0]
    valid = pos < E
    pos_c = jnp.minimum(pos, E - 1)
    spread = jnp.arange(_EPAD, dtype=jnp.int32)
    src = jnp.where(valid, edge_index[0, pos_c], spread % N).astype(jnp.int32)
    dst = jnp.where(valid, edge_index[1, pos_c],
                    N + (spread % (_NACC - N))).astype(jnp.int32)
    deg = jnp.ones((N,), jnp.float32).at[jnp.minimum(dst, N - 1)].add(
        valid.astype(jnp.float32))
    return src, dst, lax.rsqrt(deg)


def _make_segsum(P, nb):
    from jax.experimental.pallas import tpu_sc as plsc

    mesh = plsc.VectorSubcoreMesh(
        core_axis_name="c", subcore_axis_name="s", num_cores=_NSC,
        num_subcores=_NSUB)
    acc_rows_per_sub = _NACC // _NSUB
    n_drain = acc_rows_per_sub // _EBATCH
    nh = 2
    nb_h = nb // nh

    @functools.partial(
        pl.kernel,
        out_type=jax.ShapeDtypeStruct((P, _NACC, _LANE), jnp.float32),
        mesh=mesh,
        scratch_types=[
            pltpu.VMEM_SHARED((_NACC, _LANE), jnp.float32),
            pltpu.VMEM((nb_h + 8, _EBATCH), jnp.int32),
            pltpu.VMEM((nb_h, _EBATCH), jnp.int32),
            pltpu.VMEM((_EBATCH, _LANE), jnp.float32),
            pltpu.VMEM((_EBATCH, _LANE), jnp.float32),
            pltpu.SemaphoreType.DMA,
            pltpu.SemaphoreType.DMA,
        ],
    )
    def segsum(table_hbm, src_hbm, dst_hbm, zeros_hbm, out_hbm,
               acc, src_v, dst_v, rows0, rows1, sem0, sem1):
        cid = lax.axis_index("c")
        sid = lax.axis_index("s")

        def process_pair(p):
            pltpu.sync_copy(zeros_hbm, rows0)
            for j in range(acc_rows_per_sub // _EBATCH):
                pltpu.sync_copy(
                    rows0,
                    acc.at[pl.ds(sid * acc_rows_per_sub + j * _EBATCH,
                                 _EBATCH)])
            plsc.subcore_barrier()

            tbl = table_hbm.at[p]

            def gather(b, buf, sem):
                pltpu.make_async_copy(tbl.at[src_v.at[b]], buf, sem).start()

            def wait(buf, sem):
                pltpu.make_async_copy(tbl.at[pl.ds(0, _EBATCH)], buf,
                                      sem).wait()

            for hh in range(nh):
                pltpu.sync_copy(
                    src_hbm.at[sid].at[pl.ds(hh * nb_h, nb_h + 8)], src_v)
                pltpu.sync_copy(
                    dst_hbm.at[sid].at[pl.ds(hh * nb_h, nb_h)], dst_v)

                gather(0, rows0, sem0)
                gather(1, rows1, sem1)

                def body(i, carry):
                    b0 = 2 * i
                    wait(rows0, sem0)
                    pltpu.sync_copy(rows0, acc.at[dst_v.at[b0]], add=True)
                    gather(b0 + 2, rows0, sem0)
                    wait(rows1, sem1)
                    pltpu.sync_copy(rows1, acc.at[dst_v.at[b0 + 1]],
                                    add=True)
                    gather(b0 + 3, rows1, sem1)
                    return carry

                lax.fori_loop(0, nb_h // 2, body, 0, unroll=False)
                wait(rows0, sem0)
                wait(rows1, sem1)
            plsc.subcore_barrier()
            for j in range(n_drain):
                r0 = sid * acc_rows_per_sub + j * _EBATCH
                pltpu.sync_copy(acc.at[pl.ds(r0, _EBATCH)], rows0)
                pltpu.sync_copy(rows0, out_hbm.at[p].at[pl.ds(r0, _EBATCH)])
            plsc.subcore_barrier()

        for k in range(_NSC):
            @pl.when(cid == k)
            def _run():
                for p in range(k, P, _NSC):
                    process_pair(p)

    return segsum


def _segsum_sc(table, src, dst):
    P = table.shape[0]
    nb = _EPAD // (_NSUB * _EBATCH)
    src_hbm = src.reshape(_NSUB, nb, _EBATCH)
    pad_src = jnp.arange(_NSUB * 8 * _EBATCH, dtype=jnp.int32) % N
    src_hbm = jnp.concatenate(
        [src_hbm, pad_src.reshape(_NSUB, 8, _EBATCH)], axis=1)
    dst_hbm = dst.reshape(_NSUB, nb, _EBATCH)
    zeros_hbm = jnp.zeros((_EBATCH, _LANE), jnp.float32)
    fn = _make_segsum(P, nb)
    return fn(table, src_hbm, dst_hbm, zeros_hbm)


def kernel(x, edge_index, W1, b1, a1, W2, b2, a2):
    rkey = jax.random.key(42)
    ke1, ke2, kf1, kf2 = jax.random.split(rkey, 4)
    keeps = [jax.random.bernoulli(k, 1.0 - DROP_EDGE, (E,)) for k in (ke1, ke2)]
    fmasks = [jax.random.bernoulli(k, 1.0 - DROP_FEAT, (1, D_IN))
              for k in (kf1, kf2)]

    b1r = b1[None, :]
    b2r = b2[None, :]
    a1r = a1[:, None]
    a2r = a2[:, None]

    srcs, dsts, dinvs = [], [], []
    for v in range(2):
        src, dst, dinv = _compact_edges(edge_index, keeps[v])
        srcs.append(src)
        dsts.append(dst)
        dinvs.append(dinv[:, None])
    W1vs = [W1 * fmasks[v][0].astype(jnp.float32)[:, None] for v in range(2)]

    zs = []
    for v in range(2):
        xs = _make_table(x, dinvs[v])
        S1 = _segsum_sc(xs, srcs[v], dsts[v])
        hs = _mid_layer(S1, xs, dinvs[v], W1vs[v], W2, b1r, a1r)
        S2 = _segsum_sc(hs, srcs[v], dsts[v])

        h2, stats = _post_layer(S2, hs, dinvs[v], b2r, a2r)
        mu = stats[0] / N
        var = (stats[1] - N * mu * mu) / (N - 1)
        sd = jnp.sqrt(var)
        zs.append(_standardize(h2, mu[None, :], sd[None, :]))
    return (zs[0], zs[1])
